# Initial kernel scaffold; baseline (speedup 1.0000x reference)
import jax
import jax.numpy as jnp
from jax.experimental import pallas as pl


def kernel(x0, x1, x2, x3, edge_index0, edge_index1, edge_index2, edge_index3, Wself, Wnbr, b, Wp, bp, Wv, bv):
    raise NotImplementedError("write your pallas kernel here")


# speedup vs baseline: 4.3009x; 4.3009x over previous
import dataclasses
import functools

import jax
import jax.numpy as jnp
from jax import lax
from jax.experimental import pallas as pl
from jax.experimental.pallas import tpu as pltpu
from jax.experimental.pallas import tpu_sc as plsc

N = 10000
D = 128
E = 320000
A = 16
L = 3
N_AGENTS = 4

NC = 2
NS = 16
EW = E // (NC * NS)
C = 80
NCH = EW // C
N_PAD = 10240
RPT = N_PAD // NS
ZCH = 32

_MESH = plsc.VectorSubcoreMesh(core_axis_name="c", subcore_axis_name="s")

_CP = pltpu.CompilerParams()
if "needs_layout_passes" in pltpu.CompilerParams.__dataclass_fields__:
    _CP = dataclasses.replace(_CP, needs_layout_passes=False)


def _sc_count(dst4):
    @functools.partial(
        pl.kernel,
        out_type=jax.ShapeDtypeStruct((NC, NS, N), jnp.float32),
        mesh=_MESH,
        compiler_params=_CP,
        scratch_types=[
            pltpu.VMEM((NCH, C), jnp.int32),
            pltpu.VMEM((N,), jnp.float32),
        ],
    )
    def k(dst_hbm, out_hbm, dst_v, cnt_v):
        c = lax.axis_index("c")
        s = lax.axis_index("s")
        pltpu.sync_copy(dst_hbm.at[c, s], dst_v)
        z16 = jnp.zeros((16,), jnp.float32)

        @pl.loop(0, N, step=16)
        def _(i):
            cnt_v[pl.ds(i, 16)] = z16

        ones = jnp.ones((16,), jnp.float32)

        @pl.loop(0, NCH)
        def _(j):
            @pl.loop(0, C, step=16)
            def _(t):
                idx = dst_v[j, pl.ds(t, 16)]
                plsc.addupdate_scatter(cnt_v, [idx], ones)

        pltpu.sync_copy(cnt_v, out_hbm.at[c, s])

    return k(dst4)


def _sc_segsum(h, src4, dst4):
    @functools.partial(
        pl.kernel,
        out_type=jax.ShapeDtypeStruct((NC, N_PAD, D), jnp.float32),
        mesh=_MESH,
        scratch_types=[
            pltpu.VMEM((NCH, C), jnp.int32),
            pltpu.VMEM((NCH, C), jnp.int32),
            pltpu.VMEM((C, D), jnp.float32),
            pltpu.VMEM((ZCH, D), jnp.float32),
            pltpu.VMEM_SHARED((N_PAD, D), jnp.float32),
        ],
    )
    def k(h_hbm, src_hbm, dst_hbm, out_hbm, src_v, dst_v, gbuf, zbuf, acc):
        c = lax.axis_index("c")
        s = lax.axis_index("s")
        z16 = jnp.zeros((16,), jnp.float32)

        @pl.loop(0, ZCH)
        def _(r):
            @pl.loop(0, D, step=16)
            def _(t):
                zbuf[r, pl.ds(t, 16)] = z16

        row0 = s * RPT
        for kk in range(RPT // ZCH):
            pltpu.sync_copy(zbuf, acc.at[pl.ds(row0 + ZCH * kk, ZCH)])
        plsc.subcore_barrier()

        pltpu.sync_copy(src_hbm.at[c, s], src_v)
        pltpu.sync_copy(dst_hbm.at[c, s], dst_v)

        @pl.loop(0, NCH)
        def _(j):
            pltpu.sync_copy(h_hbm.at[src_v.at[j]], gbuf)
            pltpu.sync_copy(gbuf, acc.at[dst_v.at[j]], add=True)

        plsc.subcore_barrier()
        for kk in range(RPT // ZCH):
            sl = pl.ds(row0 + ZCH * kk, ZCH)
            pltpu.sync_copy(acc.at[sl], out_hbm.at[c, sl])

    return k(h, src4, dst4)


_R = 2000


def _tc_layer(h, m, cnt_t, ws, wn, bias):
    def body(h_ref, m0_ref, m1_ref, cnt_ref, ws_ref, wn_ref, b_ref, o_ref):
        cnt = jnp.sum(cnt_ref[...], axis=1)
        inv = 1.0 / jnp.maximum(cnt, 1.0)
        mm = (m0_ref[0] + m1_ref[0]) * inv[:, None]
        acc = jnp.dot(h_ref[...], ws_ref[...], preferred_element_type=jnp.float32)
        acc = acc + jnp.dot(mm, wn_ref[...], preferred_element_type=jnp.float32)
        o_ref[...] = jnp.maximum(acc + b_ref[...], 0.0)

    return pl.pallas_call(
        body,
        grid=(N // _R,),
        in_specs=[
            pl.BlockSpec((_R, D), lambda i: (i, 0)),
            pl.BlockSpec((1, _R, D), lambda i: (0, i, 0)),
            pl.BlockSpec((1, _R, D), lambda i: (1, i, 0)),
            pl.BlockSpec((_R, NC * NS), lambda i: (i, 0)),
            pl.BlockSpec((D, D), lambda i: (0, 0)),
            pl.BlockSpec((D, D), lambda i: (0, 0)),
            pl.BlockSpec((1, D), lambda i: (0, 0)),
        ],
        out_specs=pl.BlockSpec((_R, D), lambda i: (i, 0)),
        out_shape=jax.ShapeDtypeStruct((N, D), jnp.float32),
    )(h, m, m, cnt_t, ws, wn, bias.reshape(1, D))


def _tc_heads(h, w_heads, b_heads):
    def body(h_ref, w_ref, b_ref, o_ref):
        o_ref[...] = (
            jnp.dot(h_ref[...], w_ref[...], preferred_element_type=jnp.float32)
            + b_ref[...]
        )

    return pl.pallas_call(
        body,
        grid=(N // _R,),
        in_specs=[
            pl.BlockSpec((_R, D), lambda i: (i, 0)),
            pl.BlockSpec((D, A + 1), lambda i: (0, 0)),
            pl.BlockSpec((1, A + 1), lambda i: (0, 0)),
        ],
        out_specs=pl.BlockSpec((_R, A + 1), lambda i: (i, 0)),
        out_shape=jax.ShapeDtypeStruct((N, A + 1), jnp.float32),
    )(h, w_heads, b_heads.reshape(1, A + 1))


def kernel(x0, x1, x2, x3, edge_index0, edge_index1, edge_index2, edge_index3,
           Wself, Wnbr, b, Wp, bp, Wv, bv):
    xs = [x0, x1, x2, x3]
    eis = [edge_index0, edge_index1, edge_index2, edge_index3]

    srcs, dsts, cnts = [], [], []
    for i in range(N_AGENTS):
        srcs.append(eis[i][0].reshape(NC, NS, NCH, C))
        dsts.append(eis[i][1].reshape(NC, NS, NCH, C))
    for i in range(N_AGENTS):
        cp = _sc_count(dsts[i])
        cnts.append(cp.reshape(NC * NS, N).T)

    hs = list(xs)
    for l in range(L):
        ms = [_sc_segsum(hs[i], srcs[i], dsts[i]) for i in range(N_AGENTS)]
        hs = [
            _tc_layer(hs[i], ms[i], cnts[i], Wself[i, l], Wnbr[i, l], b[i, l])
            for i in range(N_AGENTS)
        ]

    logits, values = [], []
    for i in range(N_AGENTS):
        wh = jnp.concatenate([Wp[i], Wv[i]], axis=1)
        bh = jnp.concatenate([bp[i], bv[i]], axis=0)
        out = _tc_heads(hs[i], wh, bh)
        logits.append(out[:, :A])
        values.append(out[:, A:])
    return (jnp.stack(logits, axis=0), jnp.stack(values, axis=0))
